# baseline (speedup 1.0000x reference)
import functools

import jax
import jax.numpy as jnp
from jax import lax
from jax.experimental import pallas as pl
from jax.experimental.pallas import tpu as pltpu
from jax.experimental.pallas import tpu_sc as plsc

N = 10000
E = 320000
E_REAL = E + N
NEG_SLOPE = 0.2

NC = 2
NS = 16
NW = NC * NS
L = 16

W = 128
EPAD = 331776
NPAD = 10240
NH = NPAD // 2
G = 8
NSP = NH + G
NSL = NH // NS


def _mm_body(x_ref, w_ref, o_ref):
    o_ref[...] = jnp.dot(x_ref[...], w_ref[...],
                         preferred_element_type=jnp.float32)


def _mm(x, w, bn=400):
    n, k = x.shape
    _, m = w.shape
    return pl.pallas_call(
        _mm_body,
        grid=(n // bn,),
        in_specs=[
            pl.BlockSpec((bn, k), lambda i: (i, 0)),
            pl.BlockSpec((k, m), lambda i: (0, 0)),
        ],
        out_specs=pl.BlockSpec((bn, m), lambda i: (i, 0)),
        out_shape=jax.ShapeDtypeStruct((n, m), jnp.float32),
    )(x, w)


def _k1_body(D, RL, loff, roff, xl_hbm, xr_hbm, src_hbm, dst_hbm, att_hbm,
             w_hbm, src_v, dst_v, rows_l, rows_r, att_v, w_v, sem1, sem2):
    c = lax.axis_index("c")
    s = lax.axis_index("s")
    wid = s * NC + c
    nwin = EPAD // (NW * W)
    pltpu.sync_copy(att_hbm, att_v)
    att_c = [att_v[pl.ds(k * L, L)] for k in range(D // L)]
    lanes = lax.iota(jnp.int32, L)

    def win(t, carry):
        base = (wid * nwin + t) * W
        pltpu.sync_copy(src_hbm.at[pl.ds(base, W)], src_v)
        pltpu.sync_copy(dst_hbm.at[pl.ds(base, W)], dst_v)
        cl = pltpu.async_copy(xl_hbm.at[src_v], rows_l, sem1)
        cr = pltpu.async_copy(xr_hbm.at[dst_v], rows_r, sem2)
        cl.wait()
        cr.wait()

        def grp(g, carry2):
            av = jnp.zeros((L,), jnp.float32)
            for j in range(L):
                b = g * L + j
                acc = jnp.zeros((L,), jnp.float32)
                for k in range(D // L):
                    z = (rows_l[b, pl.ds(loff + k * L, L)]
                         + rows_r[b, pl.ds(roff + k * L, L)])
                    hz = jnp.maximum(z, z * NEG_SLOPE)
                    acc = acc + hz * att_c[k]
                av = jnp.where(lanes == j, jnp.sum(acc), av)
            gi = base + g * L + lanes
            w_v[pl.ds(g * L, L)] = jnp.where(
                gi < E_REAL, jnp.exp(av), jnp.float32(0.0))
            return carry2

        lax.fori_loop(0, W // L, grp, 0)
        pltpu.sync_copy(w_v, w_hbm.at[pl.ds(base, W)])
        return carry

    lax.fori_loop(0, nwin, win, 0)


def _k1(D, RL, loff, roff, xl, xr, src, dst, att):
    mesh = plsc.VectorSubcoreMesh(core_axis_name="c", subcore_axis_name="s")
    return pl.kernel(
        functools.partial(_k1_body, D, RL, loff, roff),
        out_type=jax.ShapeDtypeStruct((EPAD,), jnp.float32),
        mesh=mesh,
        compiler_params=pltpu.CompilerParams(needs_layout_passes=False),
        scratch_types=[
            pltpu.VMEM((W,), jnp.int32),
            pltpu.VMEM((W,), jnp.int32),
            pltpu.VMEM((W, RL), jnp.float32),
            pltpu.VMEM((W, RL), jnp.float32),
            pltpu.VMEM((D,), jnp.float32),
            pltpu.VMEM((W,), jnp.float32),
            pltpu.SemaphoreType.DMA,
            pltpu.SemaphoreType.DMA,
        ],
    )(xl, xr, src, dst, att)


def _k4_body(D, roff, blen, xl_hbm, xr_hbm, src_hbm, dst_hbm, att_hbm,
             b_hbm, out_hbm, src_v, dst_v, rows_l, rows_r, att_v, w_v,
             bias_v, epi, den_sl, out_sp, den_sp, sem1, sem2):
    c = lax.axis_index("c")
    s = lax.axis_index("s")
    nwin = EPAD // (NS * W)
    abase = s * NSL
    nbase0 = c * NH
    pltpu.sync_copy(att_hbm, att_v)
    att_c = [att_v[pl.ds(k * L, L)] for k in range(D // L)]

    def zrow(r, carry):
        for k in range(128 // L):
            epi[r, pl.ds(k * L, L)] = jnp.zeros((L,), jnp.float32)
        return carry

    lax.fori_loop(0, NSL, zrow, 0)

    def zden(i, carry):
        den_sl[pl.ds(i * L, L)] = jnp.zeros((L,), jnp.float32)
        return carry

    lax.fori_loop(0, NSL // L, zden, 0)
    pltpu.sync_copy(epi, out_sp.at[pl.ds(abase, NSL)])
    pltpu.sync_copy(den_sl, den_sp.at[pl.ds(abase, NSL)])

    @pl.when(s == 0)
    def _():
        pltpu.sync_copy(den_sl.at[pl.ds(0, G)], den_sp.at[pl.ds(NH, G)])
        pltpu.sync_copy(epi.at[pl.ds(0, G)], out_sp.at[pl.ds(NH, G)])

    for k in range(128 // L):
        bias_v[pl.ds(k * L, L)] = jnp.zeros((L,), jnp.float32)
    pltpu.sync_copy(b_hbm, bias_v.at[pl.ds(0, blen)])
    plsc.subcore_barrier()

    def win(t, carry):
        base = (s * nwin + t) * W
        pltpu.sync_copy(src_hbm.at[pl.ds(base, W)], src_v)
        pltpu.sync_copy(dst_hbm.at[pl.ds(base, W)], dst_v)
        cl = pltpu.async_copy(xl_hbm.at[src_v], rows_l, sem1)
        cr = pltpu.async_copy(xr_hbm.at[dst_v], rows_r, sem2)
        cl.wait()
        cr.wait()
        lanes = lax.iota(jnp.int32, L)

        def grp(g, carry2):
            av = jnp.zeros((L,), jnp.float32)
            for j in range(L):
                b = g * L + j
                acc = jnp.zeros((L,), jnp.float32)
                for k in range(D // L):
                    z = (rows_l[b, pl.ds(k * L, L)]
                         + rows_r[b, pl.ds(roff + k * L, L)])
                    hz = jnp.maximum(z, z * NEG_SLOPE)
                    acc = acc + hz * att_c[k]
                av = jnp.where(lanes == j, jnp.sum(acc), av)
            gi = base + g * L + lanes
            wg = jnp.where(gi < E_REAL, jnp.exp(av), jnp.float32(0.0))
            w_v[pl.ds(g * L, L)] = wg
            for j in range(L):
                b = g * L + j
                wv = wg[j]
                for k in range(128 // L):
                    rows_l[b, pl.ds(k * L, L)] = (
                        rows_l[b, pl.ds(k * L, L)] * wv)
            return carry2

        lax.fori_loop(0, W // L, grp, 0)
        for k in range(W // L):
            d = dst_v[pl.ds(k * L, L)]
            local = d - nbase0
            inr = (local >= 0) & (local < NH)
            garb = NH + (d & (G - 1))
            dst_v[pl.ds(k * L, L)] = jnp.where(inr, local, garb)
        pltpu.sync_copy(rows_l, out_sp.at[dst_v], add=True)
        pltpu.sync_copy(w_v, den_sp.at[dst_v], add=True)
        return carry

    lax.fori_loop(0, nwin, win, 0)
    plsc.subcore_barrier()

    pltpu.sync_copy(out_sp.at[pl.ds(abase, NSL)], epi)
    pltpu.sync_copy(den_sp.at[pl.ds(abase, NSL)], den_sl)

    def ngrp(g, carry):
        dg = den_sl[pl.ds(g * L, L)]
        for j in range(L):
            r = g * L + j
            d = dg[j] + jnp.float32(1e-16)
            for k in range(128 // L):
                v = epi[r, pl.ds(k * L, L)] / d + bias_v[pl.ds(k * L, L)]
                epi[r, pl.ds(k * L, L)] = jnp.maximum(v, jnp.float32(0.0))
        return carry

    lax.fori_loop(0, NSL // L, ngrp, 0)
    pltpu.sync_copy(epi, out_hbm.at[c, pl.ds(abase, NSL)])


def _k4(D, roff, blen, xl, xr, src, dst, att, b):
    mesh = plsc.VectorSubcoreMesh(core_axis_name="c", subcore_axis_name="s")
    return pl.kernel(
        functools.partial(_k4_body, D, roff, blen),
        out_type=jax.ShapeDtypeStruct((2, NH, 128), jnp.float32),
        mesh=mesh,
        compiler_params=pltpu.CompilerParams(needs_layout_passes=False),
        scratch_types=[
            pltpu.VMEM((W,), jnp.int32),
            pltpu.VMEM((W,), jnp.int32),
            pltpu.VMEM((W, 128), jnp.float32),
            pltpu.VMEM((W, 128), jnp.float32),
            pltpu.VMEM((D,), jnp.float32),
            pltpu.VMEM((W,), jnp.float32),
            pltpu.VMEM((128,), jnp.float32),
            pltpu.VMEM((NSL, 128), jnp.float32),
            pltpu.VMEM((NSL,), jnp.float32),
            pltpu.VMEM_SHARED((NSP, 128), jnp.float32),
            pltpu.VMEM_SHARED((NSP,), jnp.float32),
            pltpu.SemaphoreType.DMA,
            pltpu.SemaphoreType.DMA,
        ],
    )(xl, xr, src, dst, att, b)


def _k2_body(fsplit, blen, nbase0, xs_hbm, src_hbm, dst_hbm, w_hbm, b_hbm,
             out_hbm, idx_v, dst_v, w_v, rows, bias_v, epi, den_sl,
             out_sp, den_sp, sem):
    c = lax.axis_index("c")
    s = lax.axis_index("s")
    nwin = EPAD // (NS * W)
    abase = s * NSL

    def zrow(r, carry):
        for k in range(128 // L):
            epi[r, pl.ds(k * L, L)] = jnp.zeros((L,), jnp.float32)
        return carry

    lax.fori_loop(0, NSL, zrow, 0)

    def zden(i, carry):
        den_sl[pl.ds(i * L, L)] = jnp.zeros((L,), jnp.float32)
        return carry

    lax.fori_loop(0, NSL // L, zden, 0)
    pltpu.sync_copy(epi, out_sp.at[pl.ds(abase, NSL)])
    pltpu.sync_copy(den_sl, den_sp.at[pl.ds(abase, NSL)])

    @pl.when(s == 0)
    def _():
        pltpu.sync_copy(den_sl.at[pl.ds(0, G)], den_sp.at[pl.ds(NH, G)])
        pltpu.sync_copy(epi.at[pl.ds(0, G)], out_sp.at[pl.ds(NH, G)])

    for k in range(128 // L):
        bias_v[pl.ds(k * L, L)] = jnp.zeros((L,), jnp.float32)
    if fsplit:
        pltpu.sync_copy(b_hbm.at[pl.ds(c * 128, 128)], bias_v)
    else:
        pltpu.sync_copy(b_hbm, bias_v.at[pl.ds(0, blen)])
    plsc.subcore_barrier()

    def win(t, carry):
        base = (s * nwin + t) * W
        pltpu.sync_copy(src_hbm.at[pl.ds(base, W)], idx_v)
        pltpu.sync_copy(dst_hbm.at[pl.ds(base, W)], dst_v)
        pltpu.sync_copy(w_hbm.at[pl.ds(base, W)], w_v)
        for k in range(W // L):
            if fsplit:
                idx_v[pl.ds(k * L, L)] = idx_v[pl.ds(k * L, L)] + c * N
            d = dst_v[pl.ds(k * L, L)]
            local = d - nbase0
            inr = (local >= 0) & (local < NH)
            garb = NH + (d & (G - 1))
            dst_v[pl.ds(k * L, L)] = jnp.where(inr, local, garb)
        pltpu.async_copy(xs_hbm.at[idx_v], rows, sem).wait()

        def egrp(g, carry2):
            wg = w_v[pl.ds(g * L, L)]
            for j in range(L):
                b = g * L + j
                wv = wg[j]
                for k in range(128 // L):
                    rows[b, pl.ds(k * L, L)] = rows[b, pl.ds(k * L, L)] * wv
            return carry2

        lax.fori_loop(0, W // L, egrp, 0)
        pltpu.sync_copy(rows, out_sp.at[dst_v], add=True)
        pltpu.sync_copy(w_v, den_sp.at[dst_v], add=True)
        return carry

    lax.fori_loop(0, nwin, win, 0)
    plsc.subcore_barrier()

    pltpu.sync_copy(out_sp.at[pl.ds(abase, NSL)], epi)
    pltpu.sync_copy(den_sp.at[pl.ds(abase, NSL)], den_sl)

    def ngrp(g, carry):
        dg = den_sl[pl.ds(g * L, L)]
        for j in range(L):
            r = g * L + j
            d = dg[j] + jnp.float32(1e-16)
            for k in range(128 // L):
                v = epi[r, pl.ds(k * L, L)] / d + bias_v[pl.ds(k * L, L)]
                epi[r, pl.ds(k * L, L)] = jnp.maximum(v, jnp.float32(0.0))
        return carry

    lax.fori_loop(0, NSL // L, ngrp, 0)
    pltpu.sync_copy(epi, out_hbm.at[c, pl.ds(abase, NSL)])


def _k2(fsplit, blen, nbase0, xs, src, dst, w, b):
    mesh = plsc.VectorSubcoreMesh(core_axis_name="c", subcore_axis_name="s")
    return pl.kernel(
        functools.partial(_k2_body, fsplit, blen, nbase0),
        out_type=jax.ShapeDtypeStruct((2, NH, 128), jnp.float32),
        mesh=mesh,
        compiler_params=pltpu.CompilerParams(needs_layout_passes=False),
        scratch_types=[
            pltpu.VMEM((W,), jnp.int32),
            pltpu.VMEM((W,), jnp.int32),
            pltpu.VMEM((W,), jnp.float32),
            pltpu.VMEM((W, 128), jnp.float32),
            pltpu.VMEM((128,), jnp.float32),
            pltpu.VMEM((NSL, 128), jnp.float32),
            pltpu.VMEM((NSL,), jnp.float32),
            pltpu.VMEM_SHARED((NSP, 128), jnp.float32),
            pltpu.VMEM_SHARED((NSP,), jnp.float32),
            pltpu.SemaphoreType.DMA,
        ],
    )(xs, src, dst, w, b)


def kernel(x, edge_index, edge_attr, target_node_idx,
           W1l, W1r, att1, b1, W2l, W2r, att2, b2):
    loop = jnp.arange(N, dtype=jnp.int32)
    npad = EPAD - E_REAL
    pad_src = (jnp.arange(npad, dtype=jnp.int32) * 131) % N
    pad_dst = (jnp.arange(npad, dtype=jnp.int32) * 197 + 13) % N
    src = jnp.concatenate([edge_index[0].astype(jnp.int32), loop, pad_src])
    dst = jnp.concatenate([edge_index[1].astype(jnp.int32), loop, pad_dst])

    w1 = jnp.concatenate([W1l, W1r], axis=1)
    p1 = _mm(x, w1)
    xl1 = jnp.asarray(p1[:, :256])
    xr1 = jnp.asarray(p1[:, 256:])
    xl1h = xl1.reshape(N, 2, 128).transpose(1, 0, 2).reshape(2 * N, 128)
    wv1 = _k1(256, 256, 0, 0, xl1, xr1, src, dst, att1)
    ha = _k2(True, 256, 0, xl1h, src, dst, wv1, b1)
    hb = _k2(True, 256, NH, xl1h, src, dst, wv1, b1)
    h = jnp.concatenate([
        jnp.concatenate([ha[0], ha[1]], axis=1),
        jnp.concatenate([hb[0], hb[1]], axis=1)], axis=0)[:N]

    w2 = jnp.concatenate([W2l, W2r], axis=1)
    w2 = jnp.pad(w2, ((0, 0), (0, 64)))
    p2 = _mm(h, w2)
    o2 = _k4(32, 32, 32, p2, p2, src, dst, att2, b2)
    return jnp.concatenate([o2[0], o2[1]], axis=0)[:N, :32]


# speedup vs baseline: 1.2873x; 1.2873x over previous
import functools

import jax
import jax.numpy as jnp
from jax import lax
from jax.experimental import pallas as pl
from jax.experimental.pallas import tpu as pltpu
from jax.experimental.pallas import tpu_sc as plsc

N = 10000
E = 320000
E_REAL = E + N
NEG_SLOPE = 0.2

NC = 2
NS = 16
NW = NC * NS
L = 16

W = 128
EPAD = 331776
NPAD = 10240
NH = NPAD // 2
G = 8
NSP = NH + G
NSL = NH // NS


def _mm_body(x_ref, w_ref, o_ref):
    o_ref[...] = jnp.dot(x_ref[...], w_ref[...],
                         preferred_element_type=jnp.float32)


def _mm(x, w, bn=400):
    n, k = x.shape
    _, m = w.shape
    return pl.pallas_call(
        _mm_body,
        grid=(n // bn,),
        in_specs=[
            pl.BlockSpec((bn, k), lambda i: (i, 0)),
            pl.BlockSpec((k, m), lambda i: (0, 0)),
        ],
        out_specs=pl.BlockSpec((bn, m), lambda i: (i, 0)),
        out_shape=jax.ShapeDtypeStruct((n, m), jnp.float32),
    )(x, w)


def _k1_body(D, RL, loff, roff, xl_hbm, xr_hbm, src_hbm, dst_hbm, att_hbm,
             w_hbm, src_v, dst_v, rows_l, rows_r, att_v, w_v, sem1, sem2):
    c = lax.axis_index("c")
    s = lax.axis_index("s")
    wid = s * NC + c
    nwin = EPAD // (NW * W)
    pltpu.sync_copy(att_hbm, att_v)
    lanes = lax.iota(jnp.int32, L)

    def win(t, carry):
        base = (wid * nwin + t) * W
        pltpu.sync_copy(src_hbm.at[pl.ds(base, W)], src_v)
        pltpu.sync_copy(dst_hbm.at[pl.ds(base, W)], dst_v)
        cl = pltpu.async_copy(xl_hbm.at[src_v], rows_l, sem1)
        cr = pltpu.async_copy(xr_hbm.at[dst_v], rows_r, sem2)
        cl.wait()
        cr.wait()

        def grp(g, carry2):
            def edge(j, av):
                b = g * L + j
                acc = jnp.zeros((L,), jnp.float32)
                for k in range(D // L):
                    z = (rows_l[b, pl.ds(loff + k * L, L)]
                         + rows_r[b, pl.ds(roff + k * L, L)])
                    hz = jnp.maximum(z, z * NEG_SLOPE)
                    acc = acc + hz * att_v[pl.ds(k * L, L)]
                return jnp.where(lanes == j, jnp.sum(acc), av)

            av = lax.fori_loop(0, L, edge, jnp.zeros((L,), jnp.float32))
            gi = base + g * L + lanes
            w_v[pl.ds(g * L, L)] = jnp.where(
                gi < E_REAL, jnp.exp(av), jnp.float32(0.0))
            return carry2

        lax.fori_loop(0, W // L, grp, 0)
        pltpu.sync_copy(w_v, w_hbm.at[pl.ds(base, W)])
        return carry

    lax.fori_loop(0, nwin, win, 0)


def _k1(D, RL, loff, roff, xl, xr, src, dst, att):
    mesh = plsc.VectorSubcoreMesh(core_axis_name="c", subcore_axis_name="s")
    return pl.kernel(
        functools.partial(_k1_body, D, RL, loff, roff),
        out_type=jax.ShapeDtypeStruct((EPAD,), jnp.float32),
        mesh=mesh,
        compiler_params=pltpu.CompilerParams(needs_layout_passes=False),
        scratch_types=[
            pltpu.VMEM((W,), jnp.int32),
            pltpu.VMEM((W,), jnp.int32),
            pltpu.VMEM((W, RL), jnp.float32),
            pltpu.VMEM((W, RL), jnp.float32),
            pltpu.VMEM((D,), jnp.float32),
            pltpu.VMEM((W,), jnp.float32),
            pltpu.SemaphoreType.DMA,
            pltpu.SemaphoreType.DMA,
        ],
    )(xl, xr, src, dst, att)


def _k4_body(D, roff, blen, xl_hbm, xr_hbm, src_hbm, dst_hbm, att_hbm,
             b_hbm, out_hbm, src_v, dst_v, rows_l, rows_r, att_v, w_v,
             bias_v, epi, den_sl, out_sp, den_sp, sem1, sem2):
    c = lax.axis_index("c")
    s = lax.axis_index("s")
    nwin = EPAD // (NS * W)
    abase = s * NSL
    nbase0 = c * NH
    pltpu.sync_copy(att_hbm, att_v)

    def zrow(r, carry):
        for k in range(128 // L):
            epi[r, pl.ds(k * L, L)] = jnp.zeros((L,), jnp.float32)
        return carry

    lax.fori_loop(0, NSL, zrow, 0)

    def zden(i, carry):
        den_sl[pl.ds(i * L, L)] = jnp.zeros((L,), jnp.float32)
        return carry

    lax.fori_loop(0, NSL // L, zden, 0)
    pltpu.sync_copy(epi, out_sp.at[pl.ds(abase, NSL)])
    pltpu.sync_copy(den_sl, den_sp.at[pl.ds(abase, NSL)])

    @pl.when(s == 0)
    def _():
        pltpu.sync_copy(den_sl.at[pl.ds(0, G)], den_sp.at[pl.ds(NH, G)])
        pltpu.sync_copy(epi.at[pl.ds(0, G)], out_sp.at[pl.ds(NH, G)])

    for k in range(128 // L):
        bias_v[pl.ds(k * L, L)] = jnp.zeros((L,), jnp.float32)
    pltpu.sync_copy(b_hbm, bias_v.at[pl.ds(0, blen)])
    plsc.subcore_barrier()

    def win(t, carry):
        base = (s * nwin + t) * W
        pltpu.sync_copy(src_hbm.at[pl.ds(base, W)], src_v)
        pltpu.sync_copy(dst_hbm.at[pl.ds(base, W)], dst_v)
        cl = pltpu.async_copy(xl_hbm.at[src_v], rows_l, sem1)
        cr = pltpu.async_copy(xr_hbm.at[dst_v], rows_r, sem2)
        cl.wait()
        cr.wait()
        lanes = lax.iota(jnp.int32, L)

        def grp(g, carry2):
            def edge(j, av):
                b = g * L + j
                acc = jnp.zeros((L,), jnp.float32)
                for k in range(D // L):
                    z = (rows_l[b, pl.ds(k * L, L)]
                         + rows_r[b, pl.ds(roff + k * L, L)])
                    hz = jnp.maximum(z, z * NEG_SLOPE)
                    acc = acc + hz * att_v[pl.ds(k * L, L)]
                return jnp.where(lanes == j, jnp.sum(acc), av)

            av = lax.fori_loop(0, L, edge, jnp.zeros((L,), jnp.float32))
            gi = base + g * L + lanes
            wg = jnp.where(gi < E_REAL, jnp.exp(av), jnp.float32(0.0))
            w_v[pl.ds(g * L, L)] = wg
            for j in range(L):
                b = g * L + j
                wv = wg[j]
                for k in range(128 // L):
                    rows_l[b, pl.ds(k * L, L)] = (
                        rows_l[b, pl.ds(k * L, L)] * wv)
            return carry2

        lax.fori_loop(0, W // L, grp, 0)
        for k in range(W // L):
            d = dst_v[pl.ds(k * L, L)]
            local = d - nbase0
            inr = (local >= 0) & (local < NH)
            garb = NH + (d & (G - 1))
            dst_v[pl.ds(k * L, L)] = jnp.where(inr, local, garb)
        pltpu.sync_copy(rows_l, out_sp.at[dst_v], add=True)
        pltpu.sync_copy(w_v, den_sp.at[dst_v], add=True)
        return carry

    lax.fori_loop(0, nwin, win, 0)
    plsc.subcore_barrier()

    pltpu.sync_copy(out_sp.at[pl.ds(abase, NSL)], epi)
    pltpu.sync_copy(den_sp.at[pl.ds(abase, NSL)], den_sl)

    def ngrp(g, carry):
        dg = den_sl[pl.ds(g * L, L)]
        for j in range(L):
            r = g * L + j
            d = dg[j] + jnp.float32(1e-16)
            for k in range(128 // L):
                v = epi[r, pl.ds(k * L, L)] / d + bias_v[pl.ds(k * L, L)]
                epi[r, pl.ds(k * L, L)] = jnp.maximum(v, jnp.float32(0.0))
        return carry

    lax.fori_loop(0, NSL // L, ngrp, 0)
    pltpu.sync_copy(epi, out_hbm.at[c, pl.ds(abase, NSL)])


def _k4(D, roff, blen, xl, xr, src, dst, att, b):
    mesh = plsc.VectorSubcoreMesh(core_axis_name="c", subcore_axis_name="s")
    return pl.kernel(
        functools.partial(_k4_body, D, roff, blen),
        out_type=jax.ShapeDtypeStruct((2, NH, 128), jnp.float32),
        mesh=mesh,
        compiler_params=pltpu.CompilerParams(needs_layout_passes=False),
        scratch_types=[
            pltpu.VMEM((W,), jnp.int32),
            pltpu.VMEM((W,), jnp.int32),
            pltpu.VMEM((W, 128), jnp.float32),
            pltpu.VMEM((W, 128), jnp.float32),
            pltpu.VMEM((D,), jnp.float32),
            pltpu.VMEM((W,), jnp.float32),
            pltpu.VMEM((128,), jnp.float32),
            pltpu.VMEM((NSL, 128), jnp.float32),
            pltpu.VMEM((NSL,), jnp.float32),
            pltpu.VMEM_SHARED((NSP, 128), jnp.float32),
            pltpu.VMEM_SHARED((NSP,), jnp.float32),
            pltpu.SemaphoreType.DMA,
            pltpu.SemaphoreType.DMA,
        ],
    )(xl, xr, src, dst, att, b)


def _k2_body(fsplit, blen, nbase0, xs_hbm, src_hbm, dst_hbm, w_hbm, b_hbm,
             out_hbm, idx_a, idx_b, dsv_a, dsv_b, w_a, w_b, rows_a, rows_b,
             bias_v, epi, den_sl, out_sp, den_sp,
             sg_a, sg_b, so_a, so_b, sd_a, sd_b):
    c = lax.axis_index("c")
    s = lax.axis_index("s")
    nwin = EPAD // (NS * W)
    abase = s * NSL

    def zrow(r, carry):
        for k in range(128 // L):
            epi[r, pl.ds(k * L, L)] = jnp.zeros((L,), jnp.float32)
        return carry

    lax.fori_loop(0, NSL, zrow, 0)

    def zden(i, carry):
        den_sl[pl.ds(i * L, L)] = jnp.zeros((L,), jnp.float32)
        return carry

    lax.fori_loop(0, NSL // L, zden, 0)
    pltpu.sync_copy(epi, out_sp.at[pl.ds(abase, NSL)])
    pltpu.sync_copy(den_sl, den_sp.at[pl.ds(abase, NSL)])

    @pl.when(s == 0)
    def _():
        pltpu.sync_copy(den_sl.at[pl.ds(0, G)], den_sp.at[pl.ds(NH, G)])
        pltpu.sync_copy(epi.at[pl.ds(0, G)], out_sp.at[pl.ds(NH, G)])

    for k in range(128 // L):
        bias_v[pl.ds(k * L, L)] = jnp.zeros((L,), jnp.float32)
    if fsplit:
        pltpu.sync_copy(b_hbm.at[pl.ds(c * 128, 128)], bias_v)
    else:
        pltpu.sync_copy(b_hbm, bias_v.at[pl.ds(0, blen)])
    plsc.subcore_barrier()

    bufs = ((idx_a, dsv_a, w_a, rows_a, sg_a, so_a, sd_a),
            (idx_b, dsv_b, w_b, rows_b, sg_b, so_b, sd_b))

    def sup(u, carry):
        descs = []
        for i, (idx_v, dst_v, w_v, rows, sg, so, sd) in enumerate(bufs):
            t = 2 * u + i
            base = (s * nwin + t) * W

            @pl.when(u > 0)
            def _(rows=rows, dst_v=dst_v, w_v=w_v, so=so, sd=sd):
                pltpu.make_async_copy(rows, out_sp.at[dst_v], so).wait()
                pltpu.make_async_copy(w_v, den_sp.at[dst_v], sd).wait()

            pltpu.sync_copy(src_hbm.at[pl.ds(base, W)], idx_v)
            pltpu.sync_copy(dst_hbm.at[pl.ds(base, W)], dst_v)
            pltpu.sync_copy(w_hbm.at[pl.ds(base, W)], w_v)
            for k in range(W // L):
                if fsplit:
                    idx_v[pl.ds(k * L, L)] = idx_v[pl.ds(k * L, L)] + c * N
                d = dst_v[pl.ds(k * L, L)]
                local = d - nbase0
                inr = (local >= 0) & (local < NH)
                garb = NH + (d & (G - 1))
                dst_v[pl.ds(k * L, L)] = jnp.where(inr, local, garb)
            descs.append(pltpu.async_copy(xs_hbm.at[idx_v], rows, sg))

        for i, (idx_v, dst_v, w_v, rows, sg, so, sd) in enumerate(bufs):
            descs[i].wait()

            def egrp(g, carry2, w_v=w_v, rows=rows):
                wg = w_v[pl.ds(g * L, L)]
                for j in range(L):
                    b = g * L + j
                    wv = wg[j]
                    for k in range(128 // L):
                        rows[b, pl.ds(k * L, L)] = (
                            rows[b, pl.ds(k * L, L)] * wv)
                return carry2

            lax.fori_loop(0, W // L, egrp, 0)
            pltpu.async_copy(rows, out_sp.at[dst_v], so, add=True)
            pltpu.async_copy(w_v, den_sp.at[dst_v], sd, add=True)
        return carry

    lax.fori_loop(0, nwin // 2, sup, 0)
    for (idx_v, dst_v, w_v, rows, sg, so, sd) in bufs:
        pltpu.make_async_copy(rows, out_sp.at[dst_v], so).wait()
        pltpu.make_async_copy(w_v, den_sp.at[dst_v], sd).wait()
    plsc.subcore_barrier()

    pltpu.sync_copy(out_sp.at[pl.ds(abase, NSL)], epi)
    pltpu.sync_copy(den_sp.at[pl.ds(abase, NSL)], den_sl)

    def ngrp(g, carry):
        dg = den_sl[pl.ds(g * L, L)]
        for j in range(L):
            r = g * L + j
            d = dg[j] + jnp.float32(1e-16)
            for k in range(128 // L):
                v = epi[r, pl.ds(k * L, L)] / d + bias_v[pl.ds(k * L, L)]
                epi[r, pl.ds(k * L, L)] = jnp.maximum(v, jnp.float32(0.0))
        return carry

    lax.fori_loop(0, NSL // L, ngrp, 0)
    pltpu.sync_copy(epi, out_hbm.at[c, pl.ds(abase, NSL)])


def _k2(fsplit, blen, nbase0, xs, src, dst, w, b):
    mesh = plsc.VectorSubcoreMesh(core_axis_name="c", subcore_axis_name="s")
    return pl.kernel(
        functools.partial(_k2_body, fsplit, blen, nbase0),
        out_type=jax.ShapeDtypeStruct((2, NH, 128), jnp.float32),
        mesh=mesh,
        compiler_params=pltpu.CompilerParams(needs_layout_passes=False),
        scratch_types=[
            pltpu.VMEM((W,), jnp.int32),
            pltpu.VMEM((W,), jnp.int32),
            pltpu.VMEM((W,), jnp.int32),
            pltpu.VMEM((W,), jnp.int32),
            pltpu.VMEM((W,), jnp.float32),
            pltpu.VMEM((W,), jnp.float32),
            pltpu.VMEM((W, 128), jnp.float32),
            pltpu.VMEM((W, 128), jnp.float32),
            pltpu.VMEM((128,), jnp.float32),
            pltpu.VMEM((NSL, 128), jnp.float32),
            pltpu.VMEM((NSL,), jnp.float32),
            pltpu.VMEM_SHARED((NSP, 128), jnp.float32),
            pltpu.VMEM_SHARED((NSP,), jnp.float32),
            pltpu.SemaphoreType.DMA,
            pltpu.SemaphoreType.DMA,
            pltpu.SemaphoreType.DMA,
            pltpu.SemaphoreType.DMA,
            pltpu.SemaphoreType.DMA,
            pltpu.SemaphoreType.DMA,
        ],
    )(xs, src, dst, w, b)


def kernel(x, edge_index, edge_attr, target_node_idx,
           W1l, W1r, att1, b1, W2l, W2r, att2, b2):
    loop = jnp.arange(N, dtype=jnp.int32)
    npad = EPAD - E_REAL
    pad_src = (jnp.arange(npad, dtype=jnp.int32) * 131) % N
    pad_dst = (jnp.arange(npad, dtype=jnp.int32) * 197 + 13) % N
    src = jnp.concatenate([edge_index[0].astype(jnp.int32), loop, pad_src])
    dst = jnp.concatenate([edge_index[1].astype(jnp.int32), loop, pad_dst])

    w1 = jnp.concatenate([W1l, W1r], axis=1)
    p1 = _mm(x, w1)
    xl1 = jnp.asarray(p1[:, :256])
    xr1 = jnp.asarray(p1[:, 256:])
    xl1h = xl1.reshape(N, 2, 128).transpose(1, 0, 2).reshape(2 * N, 128)
    wv1 = _k1(256, 256, 0, 0, xl1, xr1, src, dst, att1)
    ha = _k2(True, 256, 0, xl1h, src, dst, wv1, b1)
    hb = _k2(True, 256, NH, xl1h, src, dst, wv1, b1)
    h = jnp.concatenate([
        jnp.concatenate([ha[0], ha[1]], axis=1),
        jnp.concatenate([hb[0], hb[1]], axis=1)], axis=0)[:N]

    w2 = jnp.concatenate([W2l, W2r], axis=1)
    w2 = jnp.pad(w2, ((0, 0), (0, 64)))
    p2 = _mm(h, w2)
    o2 = _k4(32, 32, 32, p2, p2, src, dst, att2, b2)
    return jnp.concatenate([o2[0], o2[1]], axis=0)[:N, :32]
